# speedup vs baseline: 2.8893x; 2.8893x over previous
import jax
import jax.numpy as jnp
from jax.experimental import pallas as pl
from jax.experimental.pallas import tpu as pltpu

B = 4096
NODE_ID_DIM = 64
OUT_DIM = 128
ISS_PAD = 2048
ISS_BS = 512
CAT_PAD = 128


def _dense_body(eid_ref, iss_ref, sec_ref, catr_ref, catc_ref, nums_ref,
                ert_ref, ecty_ref, nW1_ref, nb1_ref, nW2_ref, nb2_ref,
                A_ref, ab1_ref, aW2_ref, ab2_ref, out_ref):
    f32 = jnp.float32
    iota_c = jax.lax.broadcasted_iota(jnp.int32, (B, CAT_PAD), 1)
    ones_col = jnp.full((B, 1), 1.0, f32)

    R = (catr_ref[:] == iota_c).astype(f32)
    e_rat = jnp.dot(R, ert_ref[:], preferred_element_type=f32)
    C = (catc_ref[:] == iota_c).astype(f32)
    e_cty = jnp.dot(C, ecty_ref[:], preferred_element_type=f32)

    h1 = jnp.maximum(jnp.dot(nums_ref[:], nW1_ref[:], preferred_element_type=f32)
                     + nb1_ref[:], 0.0)
    h_num = jnp.maximum(jnp.dot(h1, nW2_ref[:], preferred_element_type=f32)
                        + nb2_ref[:], 0.0)

    Y = (jnp.dot(eid_ref[:], A_ref[0:64, :], preferred_element_type=f32)
         + jnp.dot(e_rat, A_ref[64:80, :], preferred_element_type=f32)
         + jnp.dot(e_cty, A_ref[80:96, :], preferred_element_type=f32)
         + jnp.dot(h_num, A_ref[96:160, :], preferred_element_type=f32))
    Y1 = Y[:, 0:128]
    Y2 = Y[:, 128:256]
    Y3 = Y[:, 256:384]

    S = (sec_ref[:] == iota_c).astype(f32)
    sec_sums = jax.lax.dot_general(S, Y3, (((0,), (0,)), ((), ())),
                                   preferred_element_type=f32)
    sec_cnt = jax.lax.dot_general(S, ones_col, (((0,), (0,)), ((), ())),
                                  preferred_element_type=f32)
    sec_means = sec_sums / jnp.maximum(sec_cnt, 1.0)
    h_sec = jnp.dot(S, sec_means, preferred_element_type=f32)

    h_iss = jnp.zeros((B, OUT_DIM), f32)
    for k in range(ISS_PAD // ISS_BS):
        iota_k = jax.lax.broadcasted_iota(jnp.int32, (B, ISS_BS), 1) + k * ISS_BS
        Sk = (iss_ref[:] == iota_k).astype(f32)
        sums_k = jax.lax.dot_general(Sk, Y2, (((0,), (0,)), ((), ())),
                                     preferred_element_type=f32)
        cnt_k = jax.lax.dot_general(Sk, ones_col, (((0,), (0,)), ((), ())),
                                    preferred_element_type=f32)
        means_k = sums_k / jnp.maximum(cnt_k, 1.0)
        h_iss = h_iss + jnp.dot(Sk, means_k, preferred_element_type=f32)

    pre = jnp.maximum(Y1 + h_iss + h_sec + ab1_ref[:], 0.0)
    out_ref[:] = jnp.dot(pre, aW2_ref[:], preferred_element_type=f32) + ab2_ref[:]


def _dense_call(e_id, issuers, sectors, cat_rating, cat_country, nums,
                ert_pad, ecty_pad, nW1, nb1, nW2, nb2, A_comb, ab1, aW2, ab2):
    return pl.pallas_call(
        _dense_body,
        out_shape=jax.ShapeDtypeStruct((B, OUT_DIM), jnp.float32),
    )(e_id, issuers, sectors, cat_rating, cat_country, nums,
      ert_pad, ecty_pad, nW1, nb1, nW2, nb2, A_comb, ab1, aW2, ab2)


def kernel(node_ids, cat_rating, cat_country, nums, node_to_issuer, node_to_sector,
           id_emb, emb_rating, emb_country, nW1, nb1, nW2, nb2, aW1, ab1, aW2, ab2):
    e_id = jnp.take(id_emb, node_ids, axis=0)
    issuers = jnp.take(node_to_issuer, node_ids)
    sectors = jnp.take(node_to_sector, node_ids)

    ert_pad = jnp.zeros((CAT_PAD, 16), jnp.float32).at[:emb_rating.shape[0]].set(emb_rating)
    ecty_pad = jnp.zeros((CAT_PAD, 16), jnp.float32).at[:emb_country.shape[0]].set(emb_country)
    A_comb = jnp.concatenate([aW1[0:160], aW1[160:320], aW1[320:480]], axis=1)

    return _dense_call(
        e_id,
        issuers.reshape(B, 1).astype(jnp.int32),
        sectors.reshape(B, 1).astype(jnp.int32),
        cat_rating.reshape(B, 1).astype(jnp.int32),
        cat_country.reshape(B, 1).astype(jnp.int32),
        nums,
        ert_pad, ecty_pad,
        nW1, nb1.reshape(1, -1), nW2, nb2.reshape(1, -1),
        A_comb, ab1.reshape(1, -1), aW2, ab2.reshape(1, -1),
    )
